# baseline (speedup 1.0000x reference)
import functools

import jax
import jax.numpy as jnp
from jax import lax
from jax.experimental import pallas as pl
from jax.experimental.pallas import tpu as pltpu
from jax.experimental.pallas import tpu_sc as plsc

NSUB = 16
NCORE = 2
B = 128
LANES = 16


def _deg_kernel(npad, nb):
    rows_per_tile = npad // NSUB

    @functools.partial(
        pl.kernel,
        out_type=jax.ShapeDtypeStruct((npad,), jnp.float32),
        mesh=plsc.VectorSubcoreMesh(core_axis_name="c", subcore_axis_name="s"),
        compiler_params=pltpu.CompilerParams(needs_layout_passes=False),
        scratch_types=[
            pltpu.VMEM_SHARED((npad,), jnp.float32),
            pltpu.VMEM((nb, B), jnp.int32),
            pltpu.VMEM((B,), jnp.float32),
            pltpu.VMEM((rows_per_tile,), jnp.float32),
        ],
    )
    def k(col3, deg, accum, colv, ones, zbuf):
        c = lax.axis_index("c")
        s = lax.axis_index("s")

        @pl.when(c == 0)
        def _():
            pltpu.sync_copy(col3.at[s], colv)
            for j in range(B // LANES):
                ones[pl.ds(j * LANES, LANES)] = jnp.ones((LANES,), jnp.float32)
            for j in range(rows_per_tile // LANES):
                zbuf[pl.ds(j * LANES, LANES)] = jnp.zeros((LANES,), jnp.float32)
            pltpu.sync_copy(zbuf,
                            accum.at[pl.ds(s * rows_per_tile, rows_per_tile)])
            plsc.subcore_barrier()

            def body(b, _):
                pltpu.sync_copy(ones, accum.at[colv.at[b]], add=True)
                return _

            lax.fori_loop(0, nb, body, None)
            plsc.subcore_barrier()
            pltpu.sync_copy(accum.at[pl.ds(s * rows_per_tile, rows_per_tile)],
                            deg.at[pl.ds(s * rows_per_tile, rows_per_tile)])

    return k


def _hop_kernel(n, npad, nb, nchunks):
    cps = nchunks // NCORE
    rb_rows = 1000
    rb_tiles = n // rb_rows
    zrows = npad // NSUB // B
    half = nb // 2

    @functools.partial(
        pl.kernel,
        out_type=jax.ShapeDtypeStruct((nchunks, n, B), jnp.float32),
        mesh=plsc.VectorSubcoreMesh(core_axis_name="c", subcore_axis_name="s"),
        compiler_params=pltpu.CompilerParams(needs_layout_passes=False),
        scratch_types=[
            pltpu.VMEM_SHARED((npad, B), jnp.float32),
            pltpu.VMEM((half, B), jnp.int32),
            pltpu.VMEM((half, B), jnp.int32),
            pltpu.VMEM((B, B), jnp.float32),
            pltpu.VMEM((B, B), jnp.float32),
            pltpu.VMEM((1024,), jnp.float32),
            pltpu.SemaphoreType.DMA,
            pltpu.SemaphoreType.DMA,
            pltpu.SemaphoreType.DMA,
            pltpu.SemaphoreType.DMA,
        ],
    )
    def k(g, row3, col3, dinv, y, accum, rowv, colv, rows0, rows1, dv,
          sga, sgb, ssa, ssb):
        c = lax.axis_index("c")
        s = lax.axis_index("s")

        @pl.when(s < rb_tiles)
        def _():
            pltpu.sync_copy(dinv.at[pl.ds(s * rb_rows, 1024)], dv)

        for cc in range(cps):
            chunk = c * cps + cc

            def zb(b, _):
                for j in range(B // LANES):
                    rows0[b, pl.ds(j * LANES, LANES)] = jnp.zeros(
                        (LANES,), jnp.float32)
                return _

            lax.fori_loop(0, B, zb, None)
            for z in range(zrows):
                pltpu.sync_copy(
                    rows0, accum.at[pl.ds((s * zrows + z) * B, B), :])
            plsc.subcore_barrier()

            hsrc = g.at[chunk]
            for h in range(2):
                def zb1(b, _):
                    for j in range(B // LANES):
                        rows1[b, pl.ds(j * LANES, LANES)] = jnp.zeros(
                            (LANES,), jnp.float32)
                    return _

                lax.fori_loop(0, B, zb1, None)
                pltpu.sync_copy(row3.at[s, pl.ds(h * half, half)], rowv)
                pltpu.sync_copy(col3.at[s, pl.ds(h * half, half)], colv)
                pltpu.async_copy(hsrc.at[rowv.at[0]], rows0, sga)
                pltpu.async_copy(rows1, accum.at[colv.at[0]], ssb, add=True)

                def pair(p, _):
                    pltpu.make_async_copy(hsrc.at[rowv.at[2 * p]],
                                          rows0, sga).wait()
                    pltpu.make_async_copy(rows1, accum.at[colv.at[2 * p]],
                                          ssb).wait()
                    pltpu.async_copy(hsrc.at[rowv.at[2 * p + 1]], rows1, sgb)
                    pltpu.async_copy(rows0, accum.at[colv.at[2 * p]], ssa,
                                     add=True)
                    pltpu.make_async_copy(hsrc.at[rowv.at[2 * p + 1]],
                                          rows1, sgb).wait()
                    pltpu.make_async_copy(rows0, accum.at[colv.at[2 * p]],
                                          ssa).wait()

                    @pl.when(p < half // 2 - 1)
                    def _():
                        pltpu.async_copy(hsrc.at[rowv.at[2 * p + 2]],
                                         rows0, sga)

                    pltpu.async_copy(rows1, accum.at[colv.at[2 * p + 1]],
                                     ssb, add=True)
                    return _

                lax.fori_loop(0, half // 2, pair, None)
                pltpu.make_async_copy(rows1, accum.at[colv.at[half - 1]],
                                      ssb).wait()
            plsc.subcore_barrier()

            @pl.when(s < rb_tiles)
            def _():
                def piece(z, size):
                    r0 = s * rb_rows + z * B
                    pltpu.sync_copy(accum.at[pl.ds(r0, B), :], rows0)

                    def grp(gi, _):
                        dvec = dv[pl.ds(z * B + gi * LANES, LANES)]
                        dvec = dvec * dvec
                        for l in range(LANES):
                            sval = dvec[l]
                            r = gi * LANES + l
                            for j in range(B // LANES):
                                rows0[r, pl.ds(j * LANES, LANES)] = (
                                    rows0[r, pl.ds(j * LANES, LANES)] * sval)
                        return _

                    lax.fori_loop(0, B // LANES, grp, None)
                    pltpu.sync_copy(rows0.at[pl.ds(0, size), :],
                                    y.at[chunk, pl.ds(r0, size), :])

                def full(z, _):
                    piece(z, B)
                    return _

                lax.fori_loop(0, rb_rows // B, full, None)
                piece(rb_rows // B, rb_rows - (rb_rows // B) * B)

            if cc + 1 < cps:
                plsc.subcore_barrier()

    return k


def _dinv_kernel(n, npad):
    def body(deg_ref, dinv_ref, sd_ref):
        deg = deg_ref[...]
        idx = lax.broadcasted_iota(jnp.int32, (1, npad), 1)
        ok = (deg > 0.0) & (idx < n)
        dinv_ref[...] = jnp.where(ok, lax.rsqrt(jnp.maximum(deg, 1.0)), 0.0)
        sd_ref[...] = jnp.where(ok, jnp.sqrt(jnp.maximum(deg, 1.0)), 0.0)

    return pl.pallas_call(
        body,
        out_shape=[jax.ShapeDtypeStruct((1, npad), jnp.float32),
                   jax.ShapeDtypeStruct((1, npad), jnp.float32)],
    )


def _rowscale_kernel(n, c_in, rb=1000):
    def body(h_ref, dv_ref, g_ref):
        g_ref[...] = h_ref[...] * dv_ref[...].reshape(1, rb, 1)

    return pl.pallas_call(
        body,
        grid=(n // rb,),
        in_specs=[pl.BlockSpec((c_in, rb, B), lambda i: (0, i, 0)),
                  pl.BlockSpec((rb, 1), lambda i: (i, 0))],
        out_specs=pl.BlockSpec((c_in, rb, B), lambda i: (0, i, 0)),
        out_shape=jax.ShapeDtypeStruct((c_in, n, B), jnp.float32),
    )


def _layer_kernel(n, c_in, latent, final, rb=1000):
    c_out = latent // B
    grid = (n // rb,)

    def body(h0, g1, g2, g3, sd, w, bias, out):
        sdv = sd[...]
        acc = jnp.broadcast_to(bias[...], (rb, latent))
        for k, hk in enumerate((h0, g1, g2, g3)):
            for ci in range(c_in):
                blk = hk[ci]
                if k > 0:
                    blk = blk * sdv
                acc = acc + jnp.dot(blk, w[k, ci],
                                    preferred_element_type=jnp.float32)
        acc = jnp.maximum(acc, 0.0)
        if final:
            out[...] = acc
        else:
            for co in range(c_out):
                out[co] = acc[:, co * B:(co + 1) * B]

    h_spec = pl.BlockSpec((c_in, rb, B), lambda i: (0, i, 0))
    if final:
        o_spec = pl.BlockSpec((rb, latent), lambda i: (i, 0))
        o_shape = jax.ShapeDtypeStruct((n, latent), jnp.float32)
    else:
        o_spec = pl.BlockSpec((c_out, rb, B), lambda i: (0, i, 0))
        o_shape = jax.ShapeDtypeStruct((c_out, n, B), jnp.float32)
    return pl.pallas_call(
        body,
        grid=grid,
        in_specs=[h_spec, h_spec, h_spec, h_spec,
                  pl.BlockSpec((rb, 1), lambda i: (i, 0)),
                  pl.BlockSpec((4, c_in, B, latent), lambda i: (0, 0, 0, 0)),
                  pl.BlockSpec((1, latent), lambda i: (0, 0))],
        out_specs=o_spec,
        out_shape=o_shape,
    )


def kernel(x, edge_index, W0, W1, W2, b0, b1, b2):
    n, in_dim = x.shape
    e = edge_index.shape[1]
    latent = W0.shape[2]
    npad = ((n + B * NSUB) // (B * NSUB)) * (B * NSUB)
    egrain = B * NSUB * 16
    epad = ((e + egrain - 1) // egrain) * egrain
    nb = epad // (B * NSUB)

    row = edge_index[0]
    col = edge_index[1]
    rowp = jnp.concatenate([row, jnp.zeros((epad - e,), jnp.int32)])
    colp = jnp.concatenate([col, jnp.full((epad - e,), n, jnp.int32)])
    row3 = rowp.reshape(NSUB, nb, B)
    col3 = colp.reshape(NSUB, nb, B)

    c0 = in_dim // B
    c1 = latent // B
    x2 = jnp.transpose(x.reshape(n, c0, B), (1, 0, 2))
    w0 = W0.reshape(4, c0, B, latent)
    w1 = W1.reshape(4, c1, B, latent)
    w2 = W2.reshape(4, c1, B, latent)

    deg = _deg_kernel(npad, nb)(col3)
    dinv_p, sd_p = _dinv_kernel(n, npad)(deg.reshape(1, npad))
    dinv = dinv_p.reshape(npad)
    dinv_col = dinv[:n].reshape(n, 1)
    sd_col = sd_p.reshape(npad)[:n].reshape(n, 1)

    hop0 = _hop_kernel(n, npad, nb, c0)
    hop1 = _hop_kernel(n, npad, nb, c1)

    h = x2
    for li, (w, b, hop, ci) in enumerate(((w0, b0, hop0, c0),
                                          (w1, b1, hop1, c1),
                                          (w2, b2, hop1, c1))):
        g0 = _rowscale_kernel(n, ci)(h, dinv_col)
        g1 = hop(g0, row3, col3, dinv)
        g2 = hop(g1, row3, col3, dinv)
        g3 = hop(g2, row3, col3, dinv)
        final = li == 2
        h = _layer_kernel(n, ci, latent, final)(
            h, g1, g2, g3, sd_col, w, b.reshape(1, latent))
    return h


# speedup vs baseline: 1.1003x; 1.1003x over previous
import functools

import jax
import jax.numpy as jnp
from jax import lax
from jax.experimental import pallas as pl
from jax.experimental.pallas import tpu as pltpu
from jax.experimental.pallas import tpu_sc as plsc

NSUB = 16
NCORE = 2
B = 128
LANES = 16


def _deg_kernel(npad, nb):
    rows_per_tile = npad // NSUB

    @functools.partial(
        pl.kernel,
        out_type=jax.ShapeDtypeStruct((npad,), jnp.float32),
        mesh=plsc.VectorSubcoreMesh(core_axis_name="c", subcore_axis_name="s"),
        compiler_params=pltpu.CompilerParams(needs_layout_passes=False),
        scratch_types=[
            pltpu.VMEM_SHARED((npad,), jnp.float32),
            pltpu.VMEM((nb, B), jnp.int32),
            pltpu.VMEM((B,), jnp.float32),
            pltpu.VMEM((rows_per_tile,), jnp.float32),
        ],
    )
    def k(col3, deg, accum, colv, ones, zbuf):
        c = lax.axis_index("c")
        s = lax.axis_index("s")

        @pl.when(c == 0)
        def _():
            pltpu.sync_copy(col3.at[s], colv)
            for j in range(B // LANES):
                ones[pl.ds(j * LANES, LANES)] = jnp.ones((LANES,), jnp.float32)
            for j in range(rows_per_tile // LANES):
                zbuf[pl.ds(j * LANES, LANES)] = jnp.zeros((LANES,), jnp.float32)
            pltpu.sync_copy(zbuf,
                            accum.at[pl.ds(s * rows_per_tile, rows_per_tile)])
            plsc.subcore_barrier()

            def body(b, _):
                pltpu.sync_copy(ones, accum.at[colv.at[b]], add=True)
                return _

            lax.fori_loop(0, nb, body, None)
            plsc.subcore_barrier()
            pltpu.sync_copy(accum.at[pl.ds(s * rows_per_tile, rows_per_tile)],
                            deg.at[pl.ds(s * rows_per_tile, rows_per_tile)])

    return k


def _hop_kernel(n, npad, nb, nchunks):
    cps = nchunks // NCORE
    rb_rows = 1000
    rb_tiles = n // rb_rows
    zrows = npad // NSUB // B
    half = nb // 2

    @functools.partial(
        pl.kernel,
        out_type=jax.ShapeDtypeStruct((nchunks, n, B), jnp.float32),
        mesh=plsc.VectorSubcoreMesh(core_axis_name="c", subcore_axis_name="s"),
        compiler_params=pltpu.CompilerParams(needs_layout_passes=False),
        scratch_types=[
            pltpu.VMEM_SHARED((npad, B), jnp.float32),
            pltpu.VMEM((half, B), jnp.int32),
            pltpu.VMEM((half, B), jnp.int32),
            pltpu.VMEM((B, B), jnp.float32),
            pltpu.VMEM((B, B), jnp.float32),
            pltpu.VMEM((1024,), jnp.float32),
            pltpu.SemaphoreType.DMA,
            pltpu.SemaphoreType.DMA,
            pltpu.SemaphoreType.DMA,
            pltpu.SemaphoreType.DMA,
        ],
    )
    def k(g, row3, col3, dinv, y, accum, rowv, colv, rows0, rows1, dv,
          sga, sgb, ssa, ssb):
        c = lax.axis_index("c")
        s = lax.axis_index("s")

        @pl.when(s < rb_tiles)
        def _():
            pltpu.sync_copy(dinv.at[pl.ds(s * rb_rows, 1024)], dv)

        for cc in range(cps):
            chunk = c * cps + cc

            def zb(b, _):
                for j in range(B // LANES):
                    rows0[b, pl.ds(j * LANES, LANES)] = jnp.zeros(
                        (LANES,), jnp.float32)
                return _

            lax.fori_loop(0, B, zb, None)
            for z in range(zrows):
                pltpu.sync_copy(
                    rows0, accum.at[pl.ds((s * zrows + z) * B, B), :])
            plsc.subcore_barrier()

            hsrc = g.at[chunk]
            for h in range(2):
                pltpu.sync_copy(row3.at[s, pl.ds(h * half, half)], rowv)
                pltpu.sync_copy(col3.at[s, pl.ds(h * half, half)], colv)
                pltpu.async_copy(hsrc.at[rowv.at[0]], rows0, sga)

                def pair(p, _):
                    pltpu.async_copy(hsrc.at[rowv.at[2 * p + 1]], rows1, sgb)
                    pltpu.make_async_copy(hsrc.at[rowv.at[2 * p]],
                                          rows0, sga).wait()
                    pltpu.sync_copy(rows0, accum.at[colv.at[2 * p]], add=True)

                    @pl.when(p < half // 2 - 1)
                    def _():
                        pltpu.async_copy(hsrc.at[rowv.at[2 * p + 2]],
                                         rows0, sga)

                    pltpu.make_async_copy(hsrc.at[rowv.at[2 * p + 1]],
                                          rows1, sgb).wait()
                    pltpu.sync_copy(rows1, accum.at[colv.at[2 * p + 1]],
                                    add=True)
                    return _

                lax.fori_loop(0, half // 2, pair, None)
            plsc.subcore_barrier()

            @pl.when(s < rb_tiles)
            def _():
                def piece(z, size):
                    r0 = s * rb_rows + z * B
                    pltpu.sync_copy(accum.at[pl.ds(r0, B), :], rows0)

                    def grp(gi, _):
                        dvec = dv[pl.ds(z * B + gi * LANES, LANES)]
                        dvec = dvec * dvec
                        for l in range(LANES):
                            sval = dvec[l]
                            r = gi * LANES + l
                            for j in range(B // LANES):
                                rows0[r, pl.ds(j * LANES, LANES)] = (
                                    rows0[r, pl.ds(j * LANES, LANES)] * sval)
                        return _

                    lax.fori_loop(0, B // LANES, grp, None)
                    pltpu.sync_copy(rows0.at[pl.ds(0, size), :],
                                    y.at[chunk, pl.ds(r0, size), :])

                def full(z, _):
                    piece(z, B)
                    return _

                lax.fori_loop(0, rb_rows // B, full, None)
                piece(rb_rows // B, rb_rows - (rb_rows // B) * B)

            if cc + 1 < cps:
                plsc.subcore_barrier()

    return k


def _dinv_kernel(n, npad):
    def body(deg_ref, dinv_ref, sd_ref):
        deg = deg_ref[...]
        idx = lax.broadcasted_iota(jnp.int32, (1, npad), 1)
        ok = (deg > 0.0) & (idx < n)
        dinv_ref[...] = jnp.where(ok, lax.rsqrt(jnp.maximum(deg, 1.0)), 0.0)
        sd_ref[...] = jnp.where(ok, jnp.sqrt(jnp.maximum(deg, 1.0)), 0.0)

    return pl.pallas_call(
        body,
        out_shape=[jax.ShapeDtypeStruct((1, npad), jnp.float32),
                   jax.ShapeDtypeStruct((1, npad), jnp.float32)],
    )


def _rowscale_kernel(n, c_in, rb=1000):
    def body(h_ref, dv_ref, g_ref):
        g_ref[...] = h_ref[...] * dv_ref[...].reshape(1, rb, 1)

    return pl.pallas_call(
        body,
        grid=(n // rb,),
        in_specs=[pl.BlockSpec((c_in, rb, B), lambda i: (0, i, 0)),
                  pl.BlockSpec((rb, 1), lambda i: (i, 0))],
        out_specs=pl.BlockSpec((c_in, rb, B), lambda i: (0, i, 0)),
        out_shape=jax.ShapeDtypeStruct((c_in, n, B), jnp.float32),
    )


def _partial_kernel(n, c_in, latent, rb=1000):
    def body(h0, g1, sd, w, p_out):
        sdv = sd[...]
        acc = jnp.zeros((rb, latent), jnp.float32)
        for ci in range(c_in):
            acc = acc + jnp.dot(h0[ci], w[0, ci],
                                preferred_element_type=jnp.float32)
            acc = acc + jnp.dot(g1[ci] * sdv, w[1, ci],
                                preferred_element_type=jnp.float32)
        p_out[...] = acc

    h_spec = pl.BlockSpec((c_in, rb, B), lambda i: (0, i, 0))
    return pl.pallas_call(
        body,
        grid=(n // rb,),
        in_specs=[h_spec, h_spec,
                  pl.BlockSpec((rb, 1), lambda i: (i, 0)),
                  pl.BlockSpec((2, c_in, B, latent), lambda i: (0, 0, 0, 0))],
        out_specs=pl.BlockSpec((rb, latent), lambda i: (i, 0)),
        out_shape=jax.ShapeDtypeStruct((n, latent), jnp.float32),
    )


def _final_kernel(n, c_in, latent, final, rb=1000):
    c_out = latent // B

    def body(p_in, g2, g3, sd, dv, w, bias, *outs):
        sdv = sd[...]
        acc = p_in[...] + jnp.broadcast_to(bias[...], (rb, latent))
        for ci in range(c_in):
            acc = acc + jnp.dot(g2[ci] * sdv, w[0, ci],
                                preferred_element_type=jnp.float32)
            acc = acc + jnp.dot(g3[ci] * sdv, w[1, ci],
                                preferred_element_type=jnp.float32)
        acc = jnp.maximum(acc, 0.0)
        if final:
            outs[0][...] = acc
        else:
            gacc = acc * dv[...]
            for co in range(c_out):
                outs[0][co] = acc[:, co * B:(co + 1) * B]
                outs[1][co] = gacc[:, co * B:(co + 1) * B]

    h_spec = pl.BlockSpec((c_in, rb, B), lambda i: (0, i, 0))
    col_spec = pl.BlockSpec((rb, 1), lambda i: (i, 0))
    if final:
        o_specs = pl.BlockSpec((rb, latent), lambda i: (i, 0))
        o_shapes = jax.ShapeDtypeStruct((n, latent), jnp.float32)
    else:
        o_specs = [pl.BlockSpec((c_out, rb, B), lambda i: (0, i, 0))] * 2
        o_shapes = [jax.ShapeDtypeStruct((c_out, n, B), jnp.float32)] * 2
    return pl.pallas_call(
        body,
        grid=(n // rb,),
        in_specs=[pl.BlockSpec((rb, latent), lambda i: (i, 0)),
                  h_spec, h_spec, col_spec, col_spec,
                  pl.BlockSpec((2, c_in, B, latent), lambda i: (0, 0, 0, 0)),
                  pl.BlockSpec((1, latent), lambda i: (0, 0))],
        out_specs=o_specs,
        out_shape=o_shapes,
    )


def kernel(x, edge_index, W0, W1, W2, b0, b1, b2):
    n, in_dim = x.shape
    e = edge_index.shape[1]
    latent = W0.shape[2]
    npad = ((n + B * NSUB) // (B * NSUB)) * (B * NSUB)
    egrain = B * NSUB * 16
    epad = ((e + egrain - 1) // egrain) * egrain
    nb = epad // (B * NSUB)

    row = edge_index[0]
    col = edge_index[1]
    rowp = jnp.concatenate([row, jnp.zeros((epad - e,), jnp.int32)])
    colp = jnp.concatenate([col, jnp.full((epad - e,), n, jnp.int32)])
    row3 = rowp.reshape(NSUB, nb, B)
    col3 = colp.reshape(NSUB, nb, B)

    c0 = in_dim // B
    c1 = latent // B
    x2 = jnp.transpose(x.reshape(n, c0, B), (1, 0, 2))
    w0 = W0.reshape(4, c0, B, latent)
    w1 = W1.reshape(4, c1, B, latent)
    w2 = W2.reshape(4, c1, B, latent)

    deg = _deg_kernel(npad, nb)(col3)
    dinv_p, sd_p = _dinv_kernel(n, npad)(deg.reshape(1, npad))
    dinv = dinv_p.reshape(npad)
    dinv_col = dinv[:n].reshape(n, 1)
    sd_col = sd_p.reshape(npad)[:n].reshape(n, 1)

    hop0 = _hop_kernel(n, npad, nb, c0)
    hop1 = _hop_kernel(n, npad, nb, c1)

    h = x2
    g0 = _rowscale_kernel(n, c0)(x2, dinv_col)
    for li, (w, b, hop, ci) in enumerate(((w0, b0, hop0, c0),
                                          (w1, b1, hop1, c1),
                                          (w2, b2, hop1, c1))):
        g1 = hop(g0, row3, col3, dinv)
        g2 = hop(g1, row3, col3, dinv)
        p = _partial_kernel(n, ci, latent)(h, g1, sd_col, w[:2])
        g3 = hop(g2, row3, col3, dinv)
        final = li == 2
        res = _final_kernel(n, ci, latent, final)(
            p, g2, g3, sd_col, dinv_col, w[2:], b.reshape(1, latent))
        if not final:
            h, g0 = res
    return res


# speedup vs baseline: 1.1234x; 1.0210x over previous
import functools

import jax
import jax.numpy as jnp
from jax import lax
from jax.experimental import pallas as pl
from jax.experimental.pallas import tpu as pltpu
from jax.experimental.pallas import tpu_sc as plsc

NSUB = 16
NCORE = 2
B = 128
LANES = 16


def _deg_kernel(npad, nb):
    rows_per_tile = npad // NSUB
    half = nb // NCORE

    @functools.partial(
        pl.kernel,
        out_type=jax.ShapeDtypeStruct((NCORE * npad,), jnp.float32),
        mesh=plsc.VectorSubcoreMesh(core_axis_name="c", subcore_axis_name="s"),
        compiler_params=pltpu.CompilerParams(needs_layout_passes=False),
        scratch_types=[
            pltpu.VMEM_SHARED((npad,), jnp.float32),
            pltpu.VMEM((half, B), jnp.int32),
            pltpu.VMEM((B,), jnp.float32),
            pltpu.VMEM((rows_per_tile,), jnp.float32),
            pltpu.SemaphoreType.DMA,
        ],
    )
    def k(col3, deg2, accum, colv, ones, zbuf, sem):
        c = lax.axis_index("c")
        s = lax.axis_index("s")
        pltpu.sync_copy(col3.at[s, pl.ds(c * half, half)], colv)
        for j in range(B // LANES):
            ones[pl.ds(j * LANES, LANES)] = jnp.ones((LANES,), jnp.float32)
        for j in range(rows_per_tile // LANES):
            zbuf[pl.ds(j * LANES, LANES)] = jnp.zeros((LANES,), jnp.float32)
        pltpu.sync_copy(zbuf,
                        accum.at[pl.ds(s * rows_per_tile, rows_per_tile)])
        plsc.subcore_barrier()

        def body(grp, _):
            for j in range(8):
                pltpu.async_copy(ones, accum.at[colv.at[grp * 8 + j]], sem,
                                 add=True)
            for j in range(8):
                pltpu.make_async_copy(ones, accum.at[colv.at[grp * 8 + j]],
                                      sem).wait()
            return _

        lax.fori_loop(0, half // 8, body, None)
        plsc.subcore_barrier()
        pltpu.sync_copy(
            accum.at[pl.ds(s * rows_per_tile, rows_per_tile)],
            deg2.at[pl.ds(c * npad + s * rows_per_tile, rows_per_tile)])

    return k


def _hop_kernel(n, npad, nb, nchunks):
    cps = nchunks // NCORE
    rb_rows = 1000
    rb_tiles = n // rb_rows
    zrows = npad // NSUB // B
    half = nb // 2

    @functools.partial(
        pl.kernel,
        out_type=jax.ShapeDtypeStruct((nchunks, n, B), jnp.float32),
        mesh=plsc.VectorSubcoreMesh(core_axis_name="c", subcore_axis_name="s"),
        compiler_params=pltpu.CompilerParams(needs_layout_passes=False),
        scratch_types=[
            pltpu.VMEM_SHARED((npad, B), jnp.float32),
            pltpu.VMEM((half, B), jnp.int32),
            pltpu.VMEM((half, B), jnp.int32),
            pltpu.VMEM((B, B), jnp.float32),
            pltpu.VMEM((B, B), jnp.float32),
            pltpu.VMEM((1024,), jnp.float32),
            pltpu.SemaphoreType.DMA,
            pltpu.SemaphoreType.DMA,
        ],
    )
    def k(g, row3, col3, dinv, y, accum, rowv, colv, rows0, rows1, dv,
          sga, sgb):
        c = lax.axis_index("c")
        s = lax.axis_index("s")

        @pl.when(s < rb_tiles)
        def _():
            pltpu.sync_copy(dinv.at[pl.ds(s * rb_rows, 1024)], dv)

        for cc in range(cps):
            chunk = c * cps + cc

            def zb(b, _):
                for j in range(B // LANES):
                    rows0[b, pl.ds(j * LANES, LANES)] = jnp.zeros(
                        (LANES,), jnp.float32)
                return _

            lax.fori_loop(0, B, zb, None)
            for z in range(zrows):
                pltpu.sync_copy(
                    rows0, accum.at[pl.ds((s * zrows + z) * B, B), :])
            plsc.subcore_barrier()

            hsrc = g.at[chunk]
            for h in range(2):
                pltpu.sync_copy(row3.at[s, pl.ds(h * half, half)], rowv)
                pltpu.sync_copy(col3.at[s, pl.ds(h * half, half)], colv)
                pltpu.async_copy(hsrc.at[rowv.at[0]], rows0, sga)

                def pair(p, _):
                    pltpu.async_copy(hsrc.at[rowv.at[2 * p + 1]], rows1, sgb)
                    pltpu.make_async_copy(hsrc.at[rowv.at[2 * p]],
                                          rows0, sga).wait()
                    pltpu.sync_copy(rows0, accum.at[colv.at[2 * p]], add=True)

                    @pl.when(p < half // 2 - 1)
                    def _():
                        pltpu.async_copy(hsrc.at[rowv.at[2 * p + 2]],
                                         rows0, sga)

                    pltpu.make_async_copy(hsrc.at[rowv.at[2 * p + 1]],
                                          rows1, sgb).wait()
                    pltpu.sync_copy(rows1, accum.at[colv.at[2 * p + 1]],
                                    add=True)
                    return _

                lax.fori_loop(0, half // 2, pair, None)
            plsc.subcore_barrier()

            @pl.when(s < rb_tiles)
            def _():
                def piece(z, size):
                    r0 = s * rb_rows + z * B
                    pltpu.sync_copy(accum.at[pl.ds(r0, B), :], rows0)

                    def grp(gi, _):
                        dvec = dv[pl.ds(z * B + gi * LANES, LANES)]
                        dvec = dvec * dvec
                        for l in range(LANES):
                            sval = dvec[l]
                            r = gi * LANES + l
                            for j in range(B // LANES):
                                rows0[r, pl.ds(j * LANES, LANES)] = (
                                    rows0[r, pl.ds(j * LANES, LANES)] * sval)
                        return _

                    lax.fori_loop(0, B // LANES, grp, None)
                    pltpu.sync_copy(rows0.at[pl.ds(0, size), :],
                                    y.at[chunk, pl.ds(r0, size), :])

                def full(z, _):
                    piece(z, B)
                    return _

                lax.fori_loop(0, rb_rows // B, full, None)
                piece(rb_rows // B, rb_rows - (rb_rows // B) * B)

            if cc + 1 < cps:
                plsc.subcore_barrier()

    return k


def _dinv_kernel(n, npad):
    def body(deg_ref, dinv_ref, sd_ref):
        deg = deg_ref[0:1, :] + deg_ref[1:2, :]
        idx = lax.broadcasted_iota(jnp.int32, (1, npad), 1)
        ok = (deg > 0.0) & (idx < n)
        dinv_ref[...] = jnp.where(ok, lax.rsqrt(jnp.maximum(deg, 1.0)), 0.0)
        sd_ref[...] = jnp.where(ok, jnp.sqrt(jnp.maximum(deg, 1.0)), 0.0)

    return pl.pallas_call(
        body,
        in_specs=[pl.BlockSpec((NCORE, npad), lambda: (0, 0))],
        out_shape=[jax.ShapeDtypeStruct((1, npad), jnp.float32),
                   jax.ShapeDtypeStruct((1, npad), jnp.float32)],
    )


def _rowscale_kernel(n, c_in, rb=1000):
    def body(h_ref, dv_ref, g_ref):
        g_ref[...] = h_ref[...] * dv_ref[...].reshape(1, rb, 1)

    return pl.pallas_call(
        body,
        grid=(n // rb,),
        in_specs=[pl.BlockSpec((c_in, rb, B), lambda i: (0, i, 0)),
                  pl.BlockSpec((rb, 1), lambda i: (i, 0))],
        out_specs=pl.BlockSpec((c_in, rb, B), lambda i: (0, i, 0)),
        out_shape=jax.ShapeDtypeStruct((c_in, n, B), jnp.float32),
    )


def _partial_kernel(n, c_in, latent, rb=1000):
    def body(h0, g1, sd, w, p_out):
        sdv = sd[...]
        acc = jnp.zeros((rb, latent), jnp.float32)
        for ci in range(c_in):
            acc = acc + jnp.dot(h0[ci], w[0, ci],
                                preferred_element_type=jnp.float32)
            acc = acc + jnp.dot(g1[ci] * sdv, w[1, ci],
                                preferred_element_type=jnp.float32)
        p_out[...] = acc

    h_spec = pl.BlockSpec((c_in, rb, B), lambda i: (0, i, 0))
    return pl.pallas_call(
        body,
        grid=(n // rb,),
        in_specs=[h_spec, h_spec,
                  pl.BlockSpec((rb, 1), lambda i: (i, 0)),
                  pl.BlockSpec((2, c_in, B, latent), lambda i: (0, 0, 0, 0))],
        out_specs=pl.BlockSpec((rb, latent), lambda i: (i, 0)),
        out_shape=jax.ShapeDtypeStruct((n, latent), jnp.float32),
    )


def _final_kernel(n, c_in, latent, final, rb=1000):
    c_out = latent // B

    def body(p_in, g2, g3, sd, dv, w, bias, *outs):
        sdv = sd[...]
        acc = p_in[...] + jnp.broadcast_to(bias[...], (rb, latent))
        for ci in range(c_in):
            acc = acc + jnp.dot(g2[ci] * sdv, w[0, ci],
                                preferred_element_type=jnp.float32)
            acc = acc + jnp.dot(g3[ci] * sdv, w[1, ci],
                                preferred_element_type=jnp.float32)
        acc = jnp.maximum(acc, 0.0)
        if final:
            outs[0][...] = acc
        else:
            gacc = acc * dv[...]
            for co in range(c_out):
                outs[0][co] = acc[:, co * B:(co + 1) * B]
                outs[1][co] = gacc[:, co * B:(co + 1) * B]

    h_spec = pl.BlockSpec((c_in, rb, B), lambda i: (0, i, 0))
    col_spec = pl.BlockSpec((rb, 1), lambda i: (i, 0))
    if final:
        o_specs = pl.BlockSpec((rb, latent), lambda i: (i, 0))
        o_shapes = jax.ShapeDtypeStruct((n, latent), jnp.float32)
    else:
        o_specs = [pl.BlockSpec((c_out, rb, B), lambda i: (0, i, 0))] * 2
        o_shapes = [jax.ShapeDtypeStruct((c_out, n, B), jnp.float32)] * 2
    return pl.pallas_call(
        body,
        grid=(n // rb,),
        in_specs=[pl.BlockSpec((rb, latent), lambda i: (i, 0)),
                  h_spec, h_spec, col_spec, col_spec,
                  pl.BlockSpec((2, c_in, B, latent), lambda i: (0, 0, 0, 0)),
                  pl.BlockSpec((1, latent), lambda i: (0, 0))],
        out_specs=o_specs,
        out_shape=o_shapes,
    )


def kernel(x, edge_index, W0, W1, W2, b0, b1, b2):
    n, in_dim = x.shape
    e = edge_index.shape[1]
    latent = W0.shape[2]
    npad = ((n + B * NSUB) // (B * NSUB)) * (B * NSUB)
    egrain = B * NSUB * 16
    epad = ((e + egrain - 1) // egrain) * egrain
    nb = epad // (B * NSUB)

    row = edge_index[0]
    col = edge_index[1]
    rowp = jnp.concatenate([row, jnp.zeros((epad - e,), jnp.int32)])
    colp = jnp.concatenate([col, jnp.full((epad - e,), n, jnp.int32)])
    row3 = rowp.reshape(NSUB, nb, B)
    col3 = colp.reshape(NSUB, nb, B)

    c0 = in_dim // B
    c1 = latent // B
    x2 = jnp.transpose(x.reshape(n, c0, B), (1, 0, 2))
    w0 = W0.reshape(4, c0, B, latent)
    w1 = W1.reshape(4, c1, B, latent)
    w2 = W2.reshape(4, c1, B, latent)

    deg2 = _deg_kernel(npad, nb)(col3)
    dinv_p, sd_p = _dinv_kernel(n, npad)(deg2.reshape(NCORE, npad))
    dinv = dinv_p.reshape(npad)
    dinv_col = dinv[:n].reshape(n, 1)
    sd_col = sd_p.reshape(npad)[:n].reshape(n, 1)

    hop0 = _hop_kernel(n, npad, nb, c0)
    hop1 = _hop_kernel(n, npad, nb, c1)

    h = x2
    g0 = _rowscale_kernel(n, c0)(x2, dinv_col)
    for li, (w, b, hop, ci) in enumerate(((w0, b0, hop0, c0),
                                          (w1, b1, hop1, c1),
                                          (w2, b2, hop1, c1))):
        g1 = hop(g0, row3, col3, dinv)
        g2 = hop(g1, row3, col3, dinv)
        p = _partial_kernel(n, ci, latent)(h, g1, sd_col, w[:2])
        g3 = hop(g2, row3, col3, dinv)
        final = li == 2
        res = _final_kernel(n, ci, latent, final)(
            p, g2, g3, sd_col, dinv_col, w[2:], b.reshape(1, latent))
        if not final:
            h, g0 = res
    return res
